# baseline (speedup 1.0000x reference)
import jax
import jax.numpy as jnp
from jax import lax
from jax.experimental import pallas as pl
from jax.experimental.pallas import tpu as pltpu
from jax.experimental.pallas import tpu_sc as plsc

_B, _D, _N, _K = 16, 256, 8192, 2048
_L = 16
_NV = _N // _L
_KV = _K // _L
_GROUP = 8
_CH = 4
_NSUB = 16


def _norm_body(se_ref, te_ref, sn_ref, tn_ref):
    se = se_ref[0]
    te = te_ref[0]
    sn_ref[0, 0, :] = lax.bitcast_convert_type(
        jnp.sqrt(jnp.sum(se * se, axis=0)), jnp.int32)
    tn_ref[0, 0, :] = lax.bitcast_convert_type(
        jnp.sqrt(jnp.sum(te * te, axis=0)), jnp.int32)


def _norms(src_embedding, tgt_embedding):
    return pl.pallas_call(
        _norm_body,
        grid=(_B,),
        in_specs=[
            pl.BlockSpec((1, _D, _N), lambda b: (b, 0, 0)),
            pl.BlockSpec((1, _D, _N), lambda b: (b, 0, 0)),
        ],
        out_specs=[
            pl.BlockSpec((1, 1, _N), lambda b: (b, 0, 0)),
            pl.BlockSpec((1, 1, _N), lambda b: (b, 0, 0)),
        ],
        out_shape=[
            jax.ShapeDtypeStruct((_B, 1, _N), jnp.int32),
            jax.ShapeDtypeStruct((_B, 1, _N), jnp.int32),
        ],
    )(src_embedding, tgt_embedding)


def _sc_body(pts_hbm, norms_hbm, s_emb_hbm, t_emb_hbm,
             kp_hbm, s_ekp_hbm, t_ekp_hbm,
             key_a, key_b, idx_a, idx_b, hist, binptr,
             rb0, rb1, rb2, rb3, obuf0, obuf1, topidx,
             sem0, sem1, sem2, sem3, sem_out0, sem_out1):
    cid = lax.axis_index("c")
    sid = lax.axis_index("s")
    wid = sid * 2 + cid
    b = jnp.where(wid < _B, wid, wid - _B)

    lane = lax.iota(jnp.int32, _L)
    ones = jnp.ones((_L,), jnp.int32)
    zeros = jnp.zeros((_L,), jnp.int32)

    pltpu.sync_copy(norms_hbm.at[pl.ds(wid, 1)], key_a)

    def radix_pass(p, src_key, src_idx, dst_key, dst_idx):
        shift = 8 * p

        def zero_body(j, _):
            hist[pl.ds(j * _L, _L)] = jnp.zeros((_L,), jnp.int32)
            return 0
        lax.fori_loop(0, 256, zero_body, 0, unroll=4)

        def hist_body(i, _):
            k = src_key[0, pl.ds(i * _L, _L)]
            d = 255 - ((k >> shift) & 255)
            plsc.addupdate_scatter(hist, [lane * 256 + d], ones)
            return 0
        lax.fori_loop(0, _NV, hist_body, 0, unroll=4)

        def pre_body(j, carry):
            tot = hist[pl.ds(j * _L, _L)]
            for l in range(1, _L):
                tot = tot + hist[pl.ds(l * 256 + j * _L, _L)]
            inc = plsc.cumsum(tot)
            binptr[pl.ds(j * _L, _L)] = inc - tot + carry
            return carry + jnp.sum(tot)
        lax.fori_loop(0, 256 // _L, pre_body, jnp.int32(0))

        def perm_body(i, _):
            k = src_key[0, pl.ds(i * _L, _L)]
            if src_idx is None:
                idv = lane + i * _L
            else:
                idv = src_idx[0, pl.ds(i * _L, _L)]
            d = 255 - ((k >> shift) & 255)
            cnt, lastm = plsc.scan_count(d)
            base = plsc.load_gather(binptr, [d])
            pos = base + cnt - 1
            plsc.store_scatter(dst_key, [zeros, pos], k)
            plsc.store_scatter(dst_idx, [zeros, pos], idv)
            plsc.addupdate_scatter(binptr, [d], cnt, mask=lastm)
            return 0
        lax.fori_loop(0, _NV, perm_body, 0, unroll=2)

    radix_pass(0, key_a, None, key_b, idx_b)
    radix_pass(1, key_b, idx_b, key_a, idx_a)
    radix_pass(2, key_a, idx_a, key_b, idx_b)
    radix_pass(3, key_b, idx_b, key_a, idx_a)

    def cp_body(i, _):
        topidx[pl.ds(i * _L, _L)] = idx_a[0, pl.ds(i * _L, _L)]
        return 0
    lax.fori_loop(0, _KV, cp_body, 0, unroll=8)

    def gather_row_to(src_rowbuf, obuf_ref, out_row):
        src1d = src_rowbuf.at[0]
        def gi(i, _):
            ids = topidx[pl.ds(i * _L, _L)]
            v = plsc.load_gather(src1d, [ids])
            obuf_ref[out_row, pl.ds(i * _L, _L)] = v
            return 0
        lax.fori_loop(0, _KV, gi, 0, unroll=8)

    for c in range(3):
        pltpu.sync_copy(pts_hbm.at[wid, pl.ds(c, 1)], rb0)
        gather_row_to(rb0, obuf0, 0)
        pltpu.sync_copy(obuf0.at[pl.ds(0, 1)], kp_hbm.at[wid, pl.ds(c, 1)])

    def emb_gather(emb_hbm, ekp_hbm):
        rbufs = (rb0, rb1, rb2, rb3)
        sems = (sem0, sem1, sem2, sem3)

        def in_copy(row, which):
            return pltpu.make_async_copy(
                emb_hbm.at[b, pl.ds(row, 1)], rbufs[which], sems[which])

        def out_copy(obuf_ref, base, sem):
            return pltpu.make_async_copy(
                obuf_ref, ekp_hbm.at[b, pl.ds(base, _GROUP)], sem)

        for w in range(3):
            in_copy(w, w).start()

        def group(g, obuf_ref, sem_out, do_wait):
            base = g * _GROUP

            @pl.when(do_wait)
            def _():
                out_copy(obuf_ref, 0, sem_out).wait()

            for rr in range(_GROUP):
                row = base + rr
                which = rr % 4
                in_copy(row, which).wait()
                nxt = row + 3

                @pl.when(nxt < _D)
                def _():
                    in_copy(nxt, (which + 3) % 4).start()
                gather_row_to(rbufs[which], obuf_ref, rr)
            out_copy(obuf_ref, base, sem_out).start()

        def gpair(gp, _):
            group(2 * gp, obuf0, sem_out0, gp > 0)
            group(2 * gp + 1, obuf1, sem_out1, gp > 0)
            return 0
        lax.fori_loop(0, _D // _GROUP // 2, gpair, 0)
        out_copy(obuf0, 0, sem_out0).wait()
        out_copy(obuf1, 0, sem_out1).wait()

    @pl.when(wid < _B)
    def _():
        emb_gather(s_emb_hbm, s_ekp_hbm)

    @pl.when(wid >= _B)
    def _():
        emb_gather(t_emb_hbm, t_ekp_hbm)


def _sc_call(pts, norms, src_embedding, tgt_embedding):
    mesh = plsc.VectorSubcoreMesh(core_axis_name="c", subcore_axis_name="s")
    f = pl.kernel(
        _sc_body,
        out_type=[
            jax.ShapeDtypeStruct((2 * _B, 3, _K), jnp.float32),
            jax.ShapeDtypeStruct((_B, _D, _K), jnp.float32),
            jax.ShapeDtypeStruct((_B, _D, _K), jnp.float32),
        ],
        mesh=mesh,
        compiler_params=pltpu.CompilerParams(needs_layout_passes=False),
        scratch_types=[
            pltpu.VMEM((1, _N), jnp.int32),
            pltpu.VMEM((1, _N), jnp.int32),
            pltpu.VMEM((1, _N), jnp.int32),
            pltpu.VMEM((1, _N), jnp.int32),
            pltpu.VMEM((256 * _L,), jnp.int32),
            pltpu.VMEM((256,), jnp.int32),
            pltpu.VMEM((1, _N), jnp.float32),
            pltpu.VMEM((1, _N), jnp.float32),
            pltpu.VMEM((1, _N), jnp.float32),
            pltpu.VMEM((1, _N), jnp.float32),
            pltpu.VMEM((_GROUP, _K), jnp.float32),
            pltpu.VMEM((_GROUP, _K), jnp.float32),
            pltpu.VMEM((_K,), jnp.int32),
            pltpu.SemaphoreType.DMA,
            pltpu.SemaphoreType.DMA,
            pltpu.SemaphoreType.DMA,
            pltpu.SemaphoreType.DMA,
            pltpu.SemaphoreType.DMA,
            pltpu.SemaphoreType.DMA,
        ],
    )
    return f(pts, norms, src_embedding, tgt_embedding)


def kernel(src, tgt, src_embedding, tgt_embedding):
    sn, tn = _norms(src_embedding, tgt_embedding)
    norms = jnp.concatenate([sn, tn], axis=0)[:, 0, :]
    pts = jnp.concatenate([src, tgt], axis=0)
    kp, s_ekp, t_ekp = _sc_call(pts, norms, src_embedding, tgt_embedding)
    return (kp[:_B], kp[_B:], s_ekp, t_ekp)


# speedup vs baseline: 1.5645x; 1.5645x over previous
import jax
import jax.numpy as jnp
from jax import lax
from jax.experimental import pallas as pl
from jax.experimental.pallas import tpu as pltpu
from jax.experimental.pallas import tpu_sc as plsc

_B, _D, _N, _K = 16, 256, 8192, 2048
_L = 16
_NV = _N // _L
_KV = _K // _L
_GROUP = 8
_CH = 4
_NSUB = 16


def _norm_body(se_ref, te_ref, sn_ref, tn_ref):
    se = se_ref[0]
    te = te_ref[0]
    sn_ref[0, 0, :] = lax.bitcast_convert_type(
        jnp.sqrt(jnp.sum(se * se, axis=0)), jnp.int32)
    tn_ref[0, 0, :] = lax.bitcast_convert_type(
        jnp.sqrt(jnp.sum(te * te, axis=0)), jnp.int32)


def _norms(src_embedding, tgt_embedding):
    return pl.pallas_call(
        _norm_body,
        grid=(_B,),
        in_specs=[
            pl.BlockSpec((1, _D, _N), lambda b: (b, 0, 0)),
            pl.BlockSpec((1, _D, _N), lambda b: (b, 0, 0)),
        ],
        out_specs=[
            pl.BlockSpec((1, 1, _N), lambda b: (b, 0, 0)),
            pl.BlockSpec((1, 1, _N), lambda b: (b, 0, 0)),
        ],
        out_shape=[
            jax.ShapeDtypeStruct((_B, 1, _N), jnp.int32),
            jax.ShapeDtypeStruct((_B, 1, _N), jnp.int32),
        ],
    )(src_embedding, tgt_embedding)


def _sc_body(pts_hbm, norms_hbm, s_emb_hbm, t_emb_hbm,
             kp_hbm, s_ekp_hbm, t_ekp_hbm,
             key_a, key_b, idx_a, idx_b, hist, binptr,
             rb0, rb1, rb2, rb3, obuf0, obuf1, topidx,
             sem0, sem1, sem2, sem3, sem_out0, sem_out1):
    cid = lax.axis_index("c")
    sid = lax.axis_index("s")
    wid = sid * 2 + cid
    b = jnp.where(wid < _B, wid, wid - _B)

    lane = lax.iota(jnp.int32, _L)
    ones = jnp.ones((_L,), jnp.int32)
    zeros = jnp.zeros((_L,), jnp.int32)

    pltpu.sync_copy(norms_hbm.at[pl.ds(wid, 1)], key_a)

    def radix_pass(p, src_key, src_idx, dst_key, dst_idx):
        shift = 8 * p

        def zero_body(j, _):
            hist[pl.ds(j * _L, _L)] = jnp.zeros((_L,), jnp.int32)
            return 0
        lax.fori_loop(0, 256, zero_body, 0, unroll=4)

        def hist_body(i, _):
            k = src_key[0, pl.ds(i * _L, _L)]
            d = 255 - ((k >> shift) & 255)
            plsc.addupdate_scatter(hist, [lane * 256 + d], ones)
            return 0
        lax.fori_loop(0, _NV, hist_body, 0, unroll=4)

        def pre_body(j, carry):
            tot = hist[pl.ds(j * _L, _L)]
            for l in range(1, _L):
                tot = tot + hist[pl.ds(l * 256 + j * _L, _L)]
            inc = plsc.cumsum(tot)
            binptr[pl.ds(j * _L, _L)] = inc - tot + carry
            return carry + jnp.sum(tot)
        lax.fori_loop(0, 256 // _L, pre_body, jnp.int32(0))

        def perm_body(i, _):
            k = src_key[0, pl.ds(i * _L, _L)]
            if src_idx is None:
                idv = lane + i * _L
            else:
                idv = src_idx[0, pl.ds(i * _L, _L)]
            d = 255 - ((k >> shift) & 255)
            cnt, lastm = plsc.scan_count(d)
            base = plsc.load_gather(binptr, [d])
            pos = base + cnt - 1
            plsc.store_scatter(dst_key, [zeros, pos], k)
            plsc.store_scatter(dst_idx, [zeros, pos], idv)
            plsc.addupdate_scatter(binptr, [d], cnt, mask=lastm)
            return 0
        lax.fori_loop(0, _NV, perm_body, 0, unroll=2)

    radix_pass(0, key_a, None, key_b, idx_b)
    radix_pass(1, key_b, idx_b, key_a, idx_a)
    radix_pass(2, key_a, idx_a, key_b, idx_b)
    radix_pass(3, key_b, idx_b, key_a, idx_a)

    @plsc.parallel_loop(0, _KV, unroll=8)
    def _(i):
        topidx[pl.ds(i * _L, _L)] = idx_a[0, pl.ds(i * _L, _L)]

    def gather_row_to(src_rowbuf, obuf_ref, out_row):
        src1d = src_rowbuf.at[0]

        @plsc.parallel_loop(0, _KV, unroll=8)
        def _(i):
            ids = topidx[pl.ds(i * _L, _L)]
            v = plsc.load_gather(src1d, [ids])
            obuf_ref[out_row, pl.ds(i * _L, _L)] = v

    for c in range(3):
        pltpu.sync_copy(pts_hbm.at[wid, pl.ds(c, 1)], rb0)
        gather_row_to(rb0, obuf0, 0)
        pltpu.sync_copy(obuf0.at[pl.ds(0, 1)], kp_hbm.at[wid, pl.ds(c, 1)])

    def emb_gather(emb_hbm, ekp_hbm):
        rbufs = (rb0, rb1, rb2, rb3)
        sems = (sem0, sem1, sem2, sem3)

        def in_copy(row, which):
            return pltpu.make_async_copy(
                emb_hbm.at[b, pl.ds(row, 1)], rbufs[which], sems[which])

        def out_copy(obuf_ref, base, sem):
            return pltpu.make_async_copy(
                obuf_ref, ekp_hbm.at[b, pl.ds(base, _GROUP)], sem)

        for w in range(3):
            in_copy(w, w).start()

        def group(g, obuf_ref, sem_out, do_wait):
            base = g * _GROUP

            @pl.when(do_wait)
            def _():
                out_copy(obuf_ref, 0, sem_out).wait()

            for rr in range(_GROUP):
                row = base + rr
                which = rr % 4
                in_copy(row, which).wait()
                nxt = row + 3

                @pl.when(nxt < _D)
                def _():
                    in_copy(nxt, (which + 3) % 4).start()
                gather_row_to(rbufs[which], obuf_ref, rr)
            out_copy(obuf_ref, base, sem_out).start()

        def gpair(gp, _):
            group(2 * gp, obuf0, sem_out0, gp > 0)
            group(2 * gp + 1, obuf1, sem_out1, gp > 0)
            return 0
        lax.fori_loop(0, _D // _GROUP // 2, gpair, 0)
        out_copy(obuf0, 0, sem_out0).wait()
        out_copy(obuf1, 0, sem_out1).wait()

    @pl.when(wid < _B)
    def _():
        emb_gather(s_emb_hbm, s_ekp_hbm)

    @pl.when(wid >= _B)
    def _():
        emb_gather(t_emb_hbm, t_ekp_hbm)


def _sc_call(pts, norms, src_embedding, tgt_embedding):
    mesh = plsc.VectorSubcoreMesh(core_axis_name="c", subcore_axis_name="s")
    f = pl.kernel(
        _sc_body,
        out_type=[
            jax.ShapeDtypeStruct((2 * _B, 3, _K), jnp.float32),
            jax.ShapeDtypeStruct((_B, _D, _K), jnp.float32),
            jax.ShapeDtypeStruct((_B, _D, _K), jnp.float32),
        ],
        mesh=mesh,
        compiler_params=pltpu.CompilerParams(needs_layout_passes=False),
        scratch_types=[
            pltpu.VMEM((1, _N), jnp.int32),
            pltpu.VMEM((1, _N), jnp.int32),
            pltpu.VMEM((1, _N), jnp.int32),
            pltpu.VMEM((1, _N), jnp.int32),
            pltpu.VMEM((256 * _L,), jnp.int32),
            pltpu.VMEM((256,), jnp.int32),
            pltpu.VMEM((1, _N), jnp.float32),
            pltpu.VMEM((1, _N), jnp.float32),
            pltpu.VMEM((1, _N), jnp.float32),
            pltpu.VMEM((1, _N), jnp.float32),
            pltpu.VMEM((_GROUP, _K), jnp.float32),
            pltpu.VMEM((_GROUP, _K), jnp.float32),
            pltpu.VMEM((_K,), jnp.int32),
            pltpu.SemaphoreType.DMA,
            pltpu.SemaphoreType.DMA,
            pltpu.SemaphoreType.DMA,
            pltpu.SemaphoreType.DMA,
            pltpu.SemaphoreType.DMA,
            pltpu.SemaphoreType.DMA,
        ],
    )
    return f(pts, norms, src_embedding, tgt_embedding)


def kernel(src, tgt, src_embedding, tgt_embedding):
    sn, tn = _norms(src_embedding, tgt_embedding)
    norms = jnp.concatenate([sn, tn], axis=0)[:, 0, :]
    pts = jnp.concatenate([src, tgt], axis=0)
    kp, s_ekp, t_ekp = _sc_call(pts, norms, src_embedding, tgt_embedding)
    return (kp[:_B], kp[_B:], s_ekp, t_ekp)


# speedup vs baseline: 1.6399x; 1.0482x over previous
import jax
import jax.numpy as jnp
from jax import lax
from jax.experimental import pallas as pl
from jax.experimental.pallas import tpu as pltpu
from jax.experimental.pallas import tpu_sc as plsc

_B, _D, _N, _K = 16, 256, 8192, 2048
_L = 16
_NV = _N // _L
_KV = _K // _L
_GROUP = 8
_CH = 4
_NSUB = 16


def _norm_body(se_ref, te_ref, sn_ref, tn_ref):
    se = se_ref[0]
    te = te_ref[0]
    sn_ref[0, 0, :] = lax.bitcast_convert_type(
        jnp.sqrt(jnp.sum(se * se, axis=0)), jnp.int32)
    tn_ref[0, 0, :] = lax.bitcast_convert_type(
        jnp.sqrt(jnp.sum(te * te, axis=0)), jnp.int32)


def _norms(src_embedding, tgt_embedding):
    return pl.pallas_call(
        _norm_body,
        grid=(_B,),
        in_specs=[
            pl.BlockSpec((1, _D, _N), lambda b: (b, 0, 0)),
            pl.BlockSpec((1, _D, _N), lambda b: (b, 0, 0)),
        ],
        out_specs=[
            pl.BlockSpec((1, 1, _N), lambda b: (b, 0, 0)),
            pl.BlockSpec((1, 1, _N), lambda b: (b, 0, 0)),
        ],
        out_shape=[
            jax.ShapeDtypeStruct((_B, 1, _N), jnp.int32),
            jax.ShapeDtypeStruct((_B, 1, _N), jnp.int32),
        ],
    )(src_embedding, tgt_embedding)


def _sc_body(pts_hbm, norms_hbm, s_emb_hbm, t_emb_hbm,
             kp_hbm, s_ekp_hbm, t_ekp_hbm,
             key_a, key_b, idx_a, idx_b, hist, binptr,
             rb0, rb1, rb2, rb3, obuf0, obuf1, topidx,
             sem0, sem1, sem2, sem3, sem_out0, sem_out1):
    cid = lax.axis_index("c")
    sid = lax.axis_index("s")
    wid = sid * 2 + cid
    b = jnp.where(wid < _B, wid, wid - _B)

    lane = lax.iota(jnp.int32, _L)
    ones = jnp.ones((_L,), jnp.int32)
    zeros = jnp.zeros((_L,), jnp.int32)

    pltpu.sync_copy(norms_hbm.at[pl.ds(wid, 1)], key_a)

    def radix_pass(p, src_key, src_idx, dst_key, dst_idx):
        shift = 8 * p

        @plsc.parallel_loop(0, 256, unroll=8)
        def _(j):
            hist[pl.ds(j * _L, _L)] = jnp.zeros((_L,), jnp.int32)

        @plsc.parallel_loop(0, _NV, unroll=8)
        def _(i):
            k = src_key[0, pl.ds(i * _L, _L)]
            d = 255 - ((k >> shift) & 255)
            plsc.addupdate_scatter(hist, [lane * 256 + d], ones)

        def pre_body(j, carry):
            tot = hist[pl.ds(j * _L, _L)]
            for l in range(1, _L):
                tot = tot + hist[pl.ds(l * 256 + j * _L, _L)]
            inc = plsc.cumsum(tot)
            binptr[pl.ds(j * _L, _L)] = inc - tot + carry
            return carry + jnp.sum(tot)
        lax.fori_loop(0, 256 // _L, pre_body, jnp.int32(0))

        def perm_body(i, _):
            k = src_key[0, pl.ds(i * _L, _L)]
            if src_idx is None:
                idv = lane + i * _L
            else:
                idv = src_idx[0, pl.ds(i * _L, _L)]
            d = 255 - ((k >> shift) & 255)
            cnt, lastm = plsc.scan_count(d)
            base = plsc.load_gather(binptr, [d])
            pos = base + cnt - 1
            if p < 3:
                plsc.store_scatter(dst_key, [zeros, pos], k)
            plsc.store_scatter(dst_idx, [zeros, pos], idv)
            plsc.addupdate_scatter(binptr, [d], cnt, mask=lastm)
            return 0
        lax.fori_loop(0, _NV, perm_body, 0, unroll=2)

    radix_pass(0, key_a, None, key_b, idx_b)
    radix_pass(1, key_b, idx_b, key_a, idx_a)
    radix_pass(2, key_a, idx_a, key_b, idx_b)
    radix_pass(3, key_b, idx_b, key_a, idx_a)

    @plsc.parallel_loop(0, _KV, unroll=8)
    def _(i):
        topidx[pl.ds(i * _L, _L)] = idx_a[0, pl.ds(i * _L, _L)]

    def gather_row_to(src_rowbuf, obuf_ref, out_row):
        src1d = src_rowbuf.at[0]

        @plsc.parallel_loop(0, _KV, unroll=8)
        def _(i):
            ids = topidx[pl.ds(i * _L, _L)]
            v = plsc.load_gather(src1d, [ids])
            obuf_ref[out_row, pl.ds(i * _L, _L)] = v

    for c in range(3):
        pltpu.sync_copy(pts_hbm.at[wid, pl.ds(c, 1)], rb0)
        gather_row_to(rb0, obuf0, 0)
        pltpu.sync_copy(obuf0.at[pl.ds(0, 1)], kp_hbm.at[wid, pl.ds(c, 1)])

    def emb_gather(emb_hbm, ekp_hbm):
        rbufs = (rb0, rb1, rb2, rb3)
        sems = (sem0, sem1, sem2, sem3)

        def in_copy(row, which):
            return pltpu.make_async_copy(
                emb_hbm.at[b, pl.ds(row, 1)], rbufs[which], sems[which])

        def out_copy(obuf_ref, base, sem):
            return pltpu.make_async_copy(
                obuf_ref, ekp_hbm.at[b, pl.ds(base, _GROUP)], sem)

        for w in range(3):
            in_copy(w, w).start()

        def group(g, obuf_ref, sem_out, do_wait):
            base = g * _GROUP

            @pl.when(do_wait)
            def _():
                out_copy(obuf_ref, 0, sem_out).wait()

            for rr in range(_GROUP):
                row = base + rr
                which = rr % 4
                in_copy(row, which).wait()
                nxt = row + 3

                @pl.when(nxt < _D)
                def _():
                    in_copy(nxt, (which + 3) % 4).start()
                gather_row_to(rbufs[which], obuf_ref, rr)
            out_copy(obuf_ref, base, sem_out).start()

        def gpair(gp, _):
            group(2 * gp, obuf0, sem_out0, gp > 0)
            group(2 * gp + 1, obuf1, sem_out1, gp > 0)
            return 0
        lax.fori_loop(0, _D // _GROUP // 2, gpair, 0)
        out_copy(obuf0, 0, sem_out0).wait()
        out_copy(obuf1, 0, sem_out1).wait()

    @pl.when(wid < _B)
    def _():
        emb_gather(s_emb_hbm, s_ekp_hbm)

    @pl.when(wid >= _B)
    def _():
        emb_gather(t_emb_hbm, t_ekp_hbm)


def _sc_call(pts, norms, src_embedding, tgt_embedding):
    mesh = plsc.VectorSubcoreMesh(core_axis_name="c", subcore_axis_name="s")
    f = pl.kernel(
        _sc_body,
        out_type=[
            jax.ShapeDtypeStruct((2 * _B, 3, _K), jnp.float32),
            jax.ShapeDtypeStruct((_B, _D, _K), jnp.float32),
            jax.ShapeDtypeStruct((_B, _D, _K), jnp.float32),
        ],
        mesh=mesh,
        compiler_params=pltpu.CompilerParams(needs_layout_passes=False),
        scratch_types=[
            pltpu.VMEM((1, _N), jnp.int32),
            pltpu.VMEM((1, _N), jnp.int32),
            pltpu.VMEM((1, _N), jnp.int32),
            pltpu.VMEM((1, _N), jnp.int32),
            pltpu.VMEM((256 * _L,), jnp.int32),
            pltpu.VMEM((256,), jnp.int32),
            pltpu.VMEM((1, _N), jnp.float32),
            pltpu.VMEM((1, _N), jnp.float32),
            pltpu.VMEM((1, _N), jnp.float32),
            pltpu.VMEM((1, _N), jnp.float32),
            pltpu.VMEM((_GROUP, _K), jnp.float32),
            pltpu.VMEM((_GROUP, _K), jnp.float32),
            pltpu.VMEM((_K,), jnp.int32),
            pltpu.SemaphoreType.DMA,
            pltpu.SemaphoreType.DMA,
            pltpu.SemaphoreType.DMA,
            pltpu.SemaphoreType.DMA,
            pltpu.SemaphoreType.DMA,
            pltpu.SemaphoreType.DMA,
        ],
    )
    return f(pts, norms, src_embedding, tgt_embedding)


def kernel(src, tgt, src_embedding, tgt_embedding):
    sn, tn = _norms(src_embedding, tgt_embedding)
    norms = jnp.concatenate([sn, tn], axis=0)[:, 0, :]
    pts = jnp.concatenate([src, tgt], axis=0)
    kp, s_ekp, t_ekp = _sc_call(pts, norms, src_embedding, tgt_embedding)
    return (kp[:_B], kp[_B:], s_ekp, t_ekp)


# speedup vs baseline: 1.7306x; 1.0553x over previous
import jax
import jax.numpy as jnp
from jax import lax
from jax.experimental import pallas as pl
from jax.experimental.pallas import tpu as pltpu
from jax.experimental.pallas import tpu_sc as plsc

_B, _D, _N, _K = 16, 256, 8192, 2048
_L = 16
_NV = _N // _L
_KV = _K // _L
_GROUP = 4
_CH = 4
_NSUB = 16


def _norm_body(se_ref, te_ref, sn_ref, tn_ref):
    se = se_ref[0]
    te = te_ref[0]
    sn_ref[0, 0, :] = lax.bitcast_convert_type(
        jnp.sqrt(jnp.sum(se * se, axis=0)), jnp.int32)
    tn_ref[0, 0, :] = lax.bitcast_convert_type(
        jnp.sqrt(jnp.sum(te * te, axis=0)), jnp.int32)


def _norms(src_embedding, tgt_embedding):
    return pl.pallas_call(
        _norm_body,
        grid=(_B,),
        in_specs=[
            pl.BlockSpec((1, _D, _N), lambda b: (b, 0, 0)),
            pl.BlockSpec((1, _D, _N), lambda b: (b, 0, 0)),
        ],
        out_specs=[
            pl.BlockSpec((1, 1, _N), lambda b: (b, 0, 0)),
            pl.BlockSpec((1, 1, _N), lambda b: (b, 0, 0)),
        ],
        out_shape=[
            jax.ShapeDtypeStruct((_B, 1, _N), jnp.int32),
            jax.ShapeDtypeStruct((_B, 1, _N), jnp.int32),
        ],
    )(src_embedding, tgt_embedding)


def _sc_body(pts_hbm, norms_hbm, s_emb_hbm, t_emb_hbm,
             kp_hbm, s_ekp_hbm, t_ekp_hbm,
             key_a, key_b, idx_a, idx_b, hist, binptr,
             rb0, rb1, rb2, rb3, obuf0, obuf1, topidx,
             sem0, sem1, sem2, sem3, sem_out0, sem_out1):
    cid = lax.axis_index("c")
    sid = lax.axis_index("s")
    wid = sid * 2 + cid
    b = jnp.where(wid < _B, wid, wid - _B)

    lane = lax.iota(jnp.int32, _L)
    ones = jnp.ones((_L,), jnp.int32)
    zeros = jnp.zeros((_L,), jnp.int32)

    pltpu.sync_copy(norms_hbm.at[pl.ds(wid, 1)], key_a)

    def radix_pass(p, src_key, src_idx, dst_key, dst_idx):
        shift = 8 * p

        @plsc.parallel_loop(0, 256, unroll=8)
        def _(j):
            hist[pl.ds(j * _L, _L)] = jnp.zeros((_L,), jnp.int32)

        @plsc.parallel_loop(0, _NV, unroll=8)
        def _(i):
            k = src_key[0, pl.ds(i * _L, _L)]
            d = 255 - ((k >> shift) & 255)
            plsc.addupdate_scatter(hist, [lane * 256 + d], ones)

        def pre_body(j, carry):
            tot = hist[pl.ds(j * _L, _L)]
            for l in range(1, _L):
                tot = tot + hist[pl.ds(l * 256 + j * _L, _L)]
            inc = plsc.cumsum(tot)
            binptr[pl.ds(j * _L, _L)] = inc - tot + carry
            return carry + jnp.sum(tot)
        lax.fori_loop(0, 256 // _L, pre_body, jnp.int32(0))

        def perm_body(i, _):
            k = src_key[0, pl.ds(i * _L, _L)]
            if src_idx is None:
                idv = lane + i * _L
            else:
                idv = src_idx[0, pl.ds(i * _L, _L)]
            d = 255 - ((k >> shift) & 255)
            cnt, lastm = plsc.scan_count(d)
            base = plsc.load_gather(binptr, [d])
            pos = base + cnt - 1
            if p < 3:
                plsc.store_scatter(dst_key, [zeros, pos], k)
            plsc.store_scatter(dst_idx, [zeros, pos], idv)
            plsc.addupdate_scatter(binptr, [d], cnt, mask=lastm)
            return 0
        lax.fori_loop(0, _NV, perm_body, 0, unroll=2)

    radix_pass(0, key_a, None, key_b, idx_b)
    radix_pass(1, key_b, idx_b, key_a, idx_a)
    radix_pass(2, key_a, idx_a, key_b, idx_b)
    radix_pass(3, key_b, idx_b, key_a, idx_a)

    @plsc.parallel_loop(0, _KV, unroll=8)
    def _(i):
        topidx[pl.ds(i * _L, _L)] = idx_a[0, pl.ds(i * _L, _L)]

    def gather_row_to(src_rowbuf, buf_row, obuf_ref, out_row):
        rowsel = zeros if buf_row == 0 else ones

        @plsc.parallel_loop(0, _KV, unroll=8)
        def _(i):
            ids = topidx[pl.ds(i * _L, _L)]
            v = plsc.load_gather(src_rowbuf, [rowsel, ids])
            obuf_ref[out_row, pl.ds(i * _L, _L)] = v

    for c in range(3):
        pltpu.sync_copy(pts_hbm.at[wid, pl.ds(c, 1)], rb0.at[pl.ds(0, 1)])
        gather_row_to(rb0, 0, obuf0, 0)
        pltpu.sync_copy(obuf0.at[pl.ds(0, 1)], kp_hbm.at[wid, pl.ds(c, 1)])

    def emb_gather(emb_hbm, ekp_hbm):
        rbufs = (rb0, rb1, rb2, rb3)
        sems = (sem0, sem1, sem2, sem3)

        def in_copy(pair, which):
            return pltpu.make_async_copy(
                emb_hbm.at[b, pl.ds(pair * 2, 2)], rbufs[which], sems[which])

        def out_copy(obuf_ref, base, sem):
            return pltpu.make_async_copy(
                obuf_ref, ekp_hbm.at[b, pl.ds(base, _GROUP)], sem)

        for w in range(3):
            in_copy(w, w).start()

        def gpair(gp, _):
            for gg in range(2):
                obuf_ref = obuf0 if gg == 0 else obuf1
                sem_out = sem_out0 if gg == 0 else sem_out1
                base = 8 * gp + 4 * gg

                @pl.when(gp > 0)
                def _():
                    out_copy(obuf_ref, 0, sem_out).wait()

                for rr in range(2):
                    which = 2 * gg + rr
                    pair = 4 * gp + which
                    in_copy(pair, which).wait()
                    nxt = pair + 3

                    @pl.when(nxt < _D // 2)
                    def _():
                        in_copy(nxt, (which + 3) % 4).start()
                    gather_row_to(rbufs[which], 0, obuf_ref, 2 * rr)
                    gather_row_to(rbufs[which], 1, obuf_ref, 2 * rr + 1)
                out_copy(obuf_ref, base, sem_out).start()
            return 0
        lax.fori_loop(0, _D // 8, gpair, 0)
        out_copy(obuf0, 0, sem_out0).wait()
        out_copy(obuf1, 0, sem_out1).wait()

    @pl.when(wid < _B)
    def _():
        emb_gather(s_emb_hbm, s_ekp_hbm)

    @pl.when(wid >= _B)
    def _():
        emb_gather(t_emb_hbm, t_ekp_hbm)


def _sc_call(pts, norms, src_embedding, tgt_embedding):
    mesh = plsc.VectorSubcoreMesh(core_axis_name="c", subcore_axis_name="s")
    f = pl.kernel(
        _sc_body,
        out_type=[
            jax.ShapeDtypeStruct((2 * _B, 3, _K), jnp.float32),
            jax.ShapeDtypeStruct((_B, _D, _K), jnp.float32),
            jax.ShapeDtypeStruct((_B, _D, _K), jnp.float32),
        ],
        mesh=mesh,
        compiler_params=pltpu.CompilerParams(needs_layout_passes=False),
        scratch_types=[
            pltpu.VMEM((1, _N), jnp.int32),
            pltpu.VMEM((1, _N), jnp.int32),
            pltpu.VMEM((1, _N), jnp.int32),
            pltpu.VMEM((1, _N), jnp.int32),
            pltpu.VMEM((256 * _L,), jnp.int32),
            pltpu.VMEM((256,), jnp.int32),
            pltpu.VMEM((2, _N), jnp.float32),
            pltpu.VMEM((2, _N), jnp.float32),
            pltpu.VMEM((2, _N), jnp.float32),
            pltpu.VMEM((2, _N), jnp.float32),
            pltpu.VMEM((_GROUP, _K), jnp.float32),
            pltpu.VMEM((_GROUP, _K), jnp.float32),
            pltpu.VMEM((_K,), jnp.int32),
            pltpu.SemaphoreType.DMA,
            pltpu.SemaphoreType.DMA,
            pltpu.SemaphoreType.DMA,
            pltpu.SemaphoreType.DMA,
            pltpu.SemaphoreType.DMA,
            pltpu.SemaphoreType.DMA,
        ],
    )
    return f(pts, norms, src_embedding, tgt_embedding)


def kernel(src, tgt, src_embedding, tgt_embedding):
    sn, tn = _norms(src_embedding, tgt_embedding)
    norms = jnp.concatenate([sn, tn], axis=0)[:, 0, :]
    pts = jnp.concatenate([src, tgt], axis=0)
    kp, s_ekp, t_ekp = _sc_call(pts, norms, src_embedding, tgt_embedding)
    return (kp[:_B], kp[_B:], s_ekp, t_ekp)
